# Initial kernel scaffold; baseline (speedup 1.0000x reference)
import jax
import jax.numpy as jnp
from jax.experimental import pallas as pl


def kernel(nodes, edges, timestamps, n_layers, user_emb, item_emb, time_emb, tables, layer_params):
    raise NotImplementedError("write your pallas kernel here")


# speedup vs baseline: 111.7565x; 111.7565x over previous
import jax
import jax.numpy as jnp
from jax import lax
from jax.experimental import pallas as pl
from jax.experimental.pallas import tpu as pltpu
from jax.experimental.pallas import tpu_sc as plsc

D = 128
H2 = D // 2
NN = 10
TPAD = 32
NC, NS = 2, 16
NW = NC * NS
NB2 = 256
NB1 = NB2 * NN

_f32 = jnp.float32
_i32 = jnp.int32


def _mesh():
    return plsc.VectorSubcoreMesh(core_axis_name="c", subcore_axis_name="s")


def _wid():
    return lax.axis_index("s") * NC + lax.axis_index("c")


def _ga_body(ptab, uemb, idxe, idxn, o_pu, o_un, ixe, ixn, bp, bu, sem):
    bpw = 4096 // NW
    base = _wid() * bpw
    pltpu.sync_copy(idxe.at[pl.ds(base, bpw)], ixe)
    pltpu.sync_copy(idxn.at[pl.ds(base, bpw)], ixn)
    pltpu.async_copy(ptab.at[ixe], bp, sem).wait()
    pltpu.sync_copy(bp, o_pu.at[pl.ds(base, bpw)])
    pltpu.async_copy(uemb.at[ixn], bu, sem).wait()
    pltpu.sync_copy(bu, o_un.at[pl.ds(base, bpw)])


def _sc_ga(ptab, uemb, idxe, idxn):
    f = pl.kernel(
        _ga_body,
        out_type=[
            jax.ShapeDtypeStruct((4096, D), _i32),
            jax.ShapeDtypeStruct((4096, D), _f32),
        ],
        mesh=_mesh(),
        scratch_types=[
            pltpu.VMEM((4096 // NW,), _i32),
            pltpu.VMEM((4096 // NW,), _i32),
            pltpu.VMEM((128, D), _i32),
            pltpu.VMEM((128, D), _f32),
            pltpu.SemaphoreType.DMA,
        ],
    )
    return f(ptab, uemb, idxe, idxn)


def _gb_body(iemb, ptab, idxa, idxp, o_ei, o_pi, ixa, ixp, bf, bi,
             sema, semb):
    bpw = 40960 // NW
    ch = 128
    base = _wid() * bpw
    pltpu.sync_copy(idxa.at[pl.ds(base, bpw)], ixa)
    pltpu.sync_copy(idxp.at[pl.ds(base, bpw)], ixp)

    def step(c, carry):
        off = c * ch
        ca = pltpu.async_copy(iemb.at[ixa.at[pl.ds(off, ch)]], bf, sema)
        cb = pltpu.async_copy(ptab.at[ixp.at[pl.ds(off, ch)]], bi, semb)
        ca.wait()
        pltpu.sync_copy(bf, o_ei.at[pl.ds(base + off, ch)])
        cb.wait()
        pltpu.sync_copy(bi, o_pi.at[pl.ds(base + off, ch)])
        return carry

    lax.fori_loop(0, bpw // ch, step, 0)


def _sc_gb(iemb, ptab, idxa, idxp):
    f = pl.kernel(
        _gb_body,
        out_type=[
            jax.ShapeDtypeStruct((40960, D), _f32),
            jax.ShapeDtypeStruct((40960, D), _i32),
        ],
        mesh=_mesh(),
        scratch_types=[
            pltpu.VMEM((40960 // NW,), _i32),
            pltpu.VMEM((40960 // NW,), _i32),
            pltpu.VMEM((128, D), _f32),
            pltpu.VMEM((128, D), _i32),
            pltpu.SemaphoreType.DMA,
            pltpu.SemaphoreType.DMA,
        ],
    )
    return f(iemb, ptab, idxa, idxp)


def _gc_body(uemb, idx, o_eu, ixv, b0, b1, s0, s1):
    bpw = 409600 // NW
    ch = 128
    nch = bpw // ch
    base = _wid() * bpw
    pltpu.sync_copy(idx.at[pl.ds(base, bpw)], ixv)
    pltpu.async_copy(uemb.at[ixv.at[pl.ds(0, ch)]], b0, s0)

    def pair(i, carry):
        off0 = (2 * i) * ch
        off1 = off0 + ch
        pltpu.async_copy(uemb.at[ixv.at[pl.ds(off1, ch)]], b1, s1)
        pltpu.make_async_copy(uemb.at[ixv.at[pl.ds(off0, ch)]], b0, s0).wait()
        pltpu.sync_copy(b0, o_eu.at[pl.ds(base + off0, ch)])

        @pl.when(i + 1 < nch // 2)
        def _():
            pltpu.async_copy(uemb.at[ixv.at[pl.ds(off1 + ch, ch)]], b0, s0)

        pltpu.make_async_copy(uemb.at[ixv.at[pl.ds(off1, ch)]], b1, s1).wait()
        pltpu.sync_copy(b1, o_eu.at[pl.ds(base + off1, ch)])
        return carry

    lax.fori_loop(0, nch // 2, pair, 0)


def _sc_gc(uemb, idx):
    f = pl.kernel(
        _gc_body,
        out_type=jax.ShapeDtypeStruct((409600, D), _f32),
        mesh=_mesh(),
        scratch_types=[
            pltpu.VMEM((409600 // NW,), _i32),
            pltpu.VMEM((128, D), _f32),
            pltpu.VMEM((128, D), _f32),
            pltpu.SemaphoreType.DMA,
            pltpu.SemaphoreType.DMA,
        ],
    )
    return f(uemb, idx)


def _tenc(delta):
    return jnp.clip(jnp.floor(delta.astype(_f32) / 5000.0), 0.0, 19.0).astype(_i32)


def _bf(x):
    return x.astype(jnp.bfloat16)


def _dot(a16, b16):
    return lax.dot_general(a16, b16, (((1,), (0,)), ((), ())),
                           preferred_element_type=_f32)


def _sel_p(n):
    rr = lax.broadcasted_iota(_i32, (D, 2 * TPAD), 0)
    mc = lax.broadcasted_iota(_i32, (D, 2 * TPAD), 1)
    cond = ((mc % TPAD) == n) & ((rr // H2) == (mc // TPAD))
    return _bf(cond)


def _dsel():
    l = lax.broadcasted_iota(_i32, (2 * TPAD, 2 * TPAD), 0)
    m = lax.broadcasted_iota(_i32, (2 * TPAD, 2 * TPAD), 1)
    return _bf((l // TPAD) == (m // TPAD))


def _pair_e(p):
    mr = lax.broadcasted_iota(_i32, (2 * TPAD, 2 * D), 0)
    dd = lax.broadcasted_iota(_i32, (2 * TPAD, 2 * D), 1)
    n_of = 2 * p + dd // D
    cond = ((mr % TPAD) == n_of) & (((dd % D) // H2) == (mr // TPAD))
    return _bf(cond)


def _attn(nodef, nb, neigh, tidx, msk, te, wq, wk, wv, f1, b1, f2, b2, vs):
    te16 = _bf(te)
    tk = _dot(te16, _bf(wk[D:]))
    tv = _dot(te16, _bf(wv[D:]))
    tq = _dot(te16[0:1], _bf(wq[D:]))
    nodef16 = _bf(nodef)
    q = _dot(nodef16, _bf(wq[:D])) + tq
    qs16 = _bf(q * 0.125)
    wkv16 = _bf(jnp.concatenate([wk[:D], wv[:D]], axis=1))
    tktv16 = jnp.concatenate([_bf(tk), _bf(tv)], axis=1)
    ohi = lax.broadcasted_iota(_i32, (nb, TPAD), 1)
    lane = lax.broadcasted_iota(_i32, (1, 2 * TPAD), 1)
    pen = jnp.where((lane % TPAD) < NN, 0.0, -1e9).astype(_f32)
    z = jnp.zeros((nb, TPAD - NN), _f32)
    mcat = jnp.concatenate([msk, z, msk, z], axis=1)

    s01 = jnp.zeros((nb, 2 * TPAD), _f32)
    for n in range(NN):
        oh16 = _bf(ohi == tidx[:, n:n + 1])
        kv = _dot(_bf(neigh(n)), wkv16) + _dot(oh16, tktv16)
        vs[pl.ds(n * nb, nb)] = kv[:, D:]
        s01 = s01 + _dot(qs16 * _bf(kv[:, :D]), _sel_p(n))

    e = jnp.exp(jnp.where(mcat > 0.5, s01, -40.0) + pen)
    d = _dot(_bf(e), _dsel())
    w16 = _bf(e * (1.0 / d))

    acc = jnp.zeros((nb, D), _f32)
    for p in range(NN // 2):
        wh = _dot(w16, _pair_e(p))
        acc = (acc + wh[:, :D] * vs[pl.ds((2 * p) * nb, nb)]
               + wh[:, D:] * vs[pl.ds((2 * p + 1) * nb, nb)])
    h = jnp.maximum(_dot(_bf(acc), _bf(f1[:D]))
                    + _dot(nodef16, _bf(f1[D:])) + b1, 0.0)
    return _dot(_bf(h), _bf(f2)) + b2


def _bc_f32(x):
    return lax.bitcast_convert_type(x, _f32)


def _tc_body(eu, ei, un, pi, pu, ts, t2p,
             te, wq0, wk0, wv0, f10, b10, f20, b20,
             wq1, wk1, wv1, f11, b11, f21, b21,
             out, o1, vs1, vs2):
    tev = te[...]
    piv = pi[...]
    tidx1 = _tenc(t2p[...] - piv[:, 50:60])
    msk1 = _bc_f32(piv[:, 60:70])
    o1[...] = _attn(ei[...], NB1, lambda n: eu[n], tidx1, msk1, tev,
                    wq0[...], wk0[...], wv0[...], f10[...], b10[...],
                    f20[...], b20[...], vs1)
    puv = pu[...]
    tidx2 = _tenc(ts[...] - puv[:, 20:30])
    msk2 = _bc_f32(puv[:, 30:40])
    out[...] = _attn(un[...], NB2, lambda n: o1[n * NB2:(n + 1) * NB2],
                     tidx2, msk2, tev,
                     wq1[...], wk1[...], wv1[...], f11[...], b11[...],
                     f21[...], b21[...], vs2)


def _tc_specs(nblk):
    full = lambda shape: pl.BlockSpec(shape, lambda i: (0,) * len(shape))
    w = [full((2 * D, D)), full((2 * D, D)), full((2 * D, D)),
         full((2 * D, D)), full((1, D)), full((D, D)), full((1, D))]
    in_specs = [
        pl.BlockSpec((NN, NB1, D), lambda i: (0, i, 0)),
        pl.BlockSpec((NB1, D), lambda i: (i, 0)),
        pl.BlockSpec((NB2, D), lambda i: (i, 0)),
        pl.BlockSpec((NB1, D), lambda i: (i, 0)),
        pl.BlockSpec((NB2, D), lambda i: (i, 0)),
        pl.BlockSpec((NB2, 1), lambda i: (i, 0)),
        pl.BlockSpec((NB1, 1), lambda i: (i, 0)),
        full((TPAD, D)),
    ] + w + w
    out_specs = pl.BlockSpec((NB2, D), lambda i: (i, 0))
    out_shape = jax.ShapeDtypeStruct((nblk * NB2, D), _f32)
    scratch = [pltpu.VMEM((NB1, D), _f32),
               pltpu.VMEM((NN * NB1, D), _f32),
               pltpu.VMEM((NN * NB2, D), _f32)]
    return dict(grid=(nblk,), in_specs=in_specs, out_specs=out_specs,
                out_shape=out_shape, scratch_shapes=scratch)


def _prep_params(p):
    return (p['Wq'], p['Wk'], p['Wv'], p['fc1_w'],
            p['fc1_b'].reshape(1, D), p['fc2_w'], p['fc2_b'].reshape(1, D))


def kernel(nodes, edges, timestamps, n_layers, user_emb, item_emb, time_emb,
           tables, layer_params):
    del n_layers
    B = nodes.shape[0]
    nblk = B // NB2
    ut, it = tables['user'], tables['item']
    ne = ut['neig'].shape[0]

    tail = lambda a: a[:, -NN:]
    bci = lambda a: lax.bitcast_convert_type(a, _i32)
    ptab = jnp.concatenate(
        [tail(ut['neig']), tail(ut['edge']), tail(ut['time']),
         bci(tail(ut['mask'])), tail(it['neig']), tail(it['time']),
         bci(tail(it['mask'])),
         jnp.zeros((ne, D - 7 * NN), _i32)], axis=1)

    pu_rows, un_nodes = _sc_ga(ptab, user_emb, edges, nodes)

    def perm(x):
        return x.reshape(nblk, NB2, NN).transpose(0, 2, 1).reshape(-1)

    adj2_p = perm(pu_rows[:, 0:NN])
    adge2_p = perm(pu_rows[:, NN:2 * NN])
    t2p = perm(pu_rows[:, 2 * NN:3 * NN]).reshape(-1, 1)
    ei, pi_rows = _sc_gb(item_emb, ptab, adj2_p, adge2_p)

    idx3 = pi_rows[:, 4 * NN:5 * NN].T.reshape(-1)
    eu = _sc_gc(user_emb, idx3).reshape(NN, B * NN, D)

    te = jnp.zeros((TPAD, D), _f32).at[:20].set(time_emb)
    p0 = _prep_params(layer_params[0])
    p1 = _prep_params(layer_params[1])
    tc = pl.pallas_call(_tc_body, **_tc_specs(nblk))
    return tc(eu, ei, un_nodes, pi_rows, pu_rows, timestamps.reshape(-1, 1),
              t2p, te, *p0, *p1)


# speedup vs baseline: 122.2090x; 1.0935x over previous
import jax
import jax.numpy as jnp
from jax import lax
from jax.experimental import pallas as pl
from jax.experimental.pallas import tpu as pltpu
from jax.experimental.pallas import tpu_sc as plsc

D = 128
H2 = D // 2
NN = 10
TPAD = 32
NC, NS = 2, 16
NW = NC * NS
NB2 = 256
NB1 = NB2 * NN

_f32 = jnp.float32
_i32 = jnp.int32


def _mesh():
    return plsc.VectorSubcoreMesh(core_axis_name="c", subcore_axis_name="s")


def _wid():
    return lax.axis_index("s") * NC + lax.axis_index("c")


def _ga_body(ptab, uemb, idxe, idxn, o_pu, o_un, ixe, ixn, bp, bu, sem):
    bpw = 4096 // NW
    base = _wid() * bpw
    pltpu.sync_copy(idxe.at[pl.ds(base, bpw)], ixe)
    pltpu.sync_copy(idxn.at[pl.ds(base, bpw)], ixn)
    pltpu.async_copy(ptab.at[ixe], bp, sem).wait()
    pltpu.sync_copy(bp, o_pu.at[pl.ds(base, bpw)])
    pltpu.async_copy(uemb.at[ixn], bu, sem).wait()
    pltpu.sync_copy(bu, o_un.at[pl.ds(base, bpw)])


def _sc_ga(ptab, uemb, idxe, idxn):
    f = pl.kernel(
        _ga_body,
        out_type=[
            jax.ShapeDtypeStruct((4096, D), _i32),
            jax.ShapeDtypeStruct((4096, D), _f32),
        ],
        mesh=_mesh(),
        scratch_types=[
            pltpu.VMEM((4096 // NW,), _i32),
            pltpu.VMEM((4096 // NW,), _i32),
            pltpu.VMEM((128, D), _i32),
            pltpu.VMEM((128, D), _f32),
            pltpu.SemaphoreType.DMA,
        ],
    )
    return f(ptab, uemb, idxe, idxn)


def _gb_body(iemb, ptab, idxa, idxp, o_ei, o_pi, ixa, ixp, bf, bi,
             sema, semb):
    bpw = 40960 // NW
    ch = 128
    base = _wid() * bpw
    pltpu.sync_copy(idxa.at[pl.ds(base, bpw)], ixa)
    pltpu.sync_copy(idxp.at[pl.ds(base, bpw)], ixp)

    def step(c, carry):
        off = c * ch
        ca = pltpu.async_copy(iemb.at[ixa.at[pl.ds(off, ch)]], bf, sema)
        cb = pltpu.async_copy(ptab.at[ixp.at[pl.ds(off, ch)]], bi, semb)
        ca.wait()
        pltpu.sync_copy(bf, o_ei.at[pl.ds(base + off, ch)])
        cb.wait()
        pltpu.sync_copy(bi, o_pi.at[pl.ds(base + off, ch)])
        return carry

    lax.fori_loop(0, bpw // ch, step, 0)


def _sc_gb(iemb, ptab, idxa, idxp):
    f = pl.kernel(
        _gb_body,
        out_type=[
            jax.ShapeDtypeStruct((40960, D), _f32),
            jax.ShapeDtypeStruct((40960, D), _i32),
        ],
        mesh=_mesh(),
        scratch_types=[
            pltpu.VMEM((40960 // NW,), _i32),
            pltpu.VMEM((40960 // NW,), _i32),
            pltpu.VMEM((128, D), _f32),
            pltpu.VMEM((128, D), _i32),
            pltpu.SemaphoreType.DMA,
            pltpu.SemaphoreType.DMA,
        ],
    )
    return f(iemb, ptab, idxa, idxp)


def _gc_body(uemb, idx, o_eu, ixv, b0, b1, s0, s1):
    bpw = 409600 // NW
    ch = 128
    nch = bpw // ch
    base = _wid() * bpw
    pltpu.sync_copy(idx.at[pl.ds(base, bpw)], ixv)
    pltpu.async_copy(uemb.at[ixv.at[pl.ds(0, ch)]], b0, s0)

    def pair(i, carry):
        off0 = (2 * i) * ch
        off1 = off0 + ch
        pltpu.async_copy(uemb.at[ixv.at[pl.ds(off1, ch)]], b1, s1)
        pltpu.make_async_copy(uemb.at[ixv.at[pl.ds(off0, ch)]], b0, s0).wait()
        pltpu.sync_copy(b0, o_eu.at[pl.ds(base + off0, ch)])

        @pl.when(i + 1 < nch // 2)
        def _():
            pltpu.async_copy(uemb.at[ixv.at[pl.ds(off1 + ch, ch)]], b0, s0)

        pltpu.make_async_copy(uemb.at[ixv.at[pl.ds(off1, ch)]], b1, s1).wait()
        pltpu.sync_copy(b1, o_eu.at[pl.ds(base + off1, ch)])
        return carry

    lax.fori_loop(0, nch // 2, pair, 0)


def _sc_gc(uemb, idx):
    f = pl.kernel(
        _gc_body,
        out_type=jax.ShapeDtypeStruct((409600, D), _f32),
        mesh=_mesh(),
        scratch_types=[
            pltpu.VMEM((409600 // NW,), _i32),
            pltpu.VMEM((128, D), _f32),
            pltpu.VMEM((128, D), _f32),
            pltpu.SemaphoreType.DMA,
            pltpu.SemaphoreType.DMA,
        ],
    )
    return f(uemb, idx)


def _tenc(delta):
    return jnp.clip(jnp.floor(delta.astype(_f32) / 5000.0), 0.0, 19.0).astype(_i32)


def _bf(x):
    return x.astype(jnp.bfloat16)


def _dot(a16, b16):
    return lax.dot_general(a16, b16, (((1,), (0,)), ((), ())),
                           preferred_element_type=_f32)


def _sel_p(n):
    rr = lax.broadcasted_iota(_i32, (D, 2 * TPAD), 0)
    mc = lax.broadcasted_iota(_i32, (D, 2 * TPAD), 1)
    cond = ((mc % TPAD) == n) & ((rr // H2) == (mc // TPAD))
    return _bf(cond)


def _dsel():
    l = lax.broadcasted_iota(_i32, (2 * TPAD, 2 * TPAD), 0)
    m = lax.broadcasted_iota(_i32, (2 * TPAD, 2 * TPAD), 1)
    return _bf((l // TPAD) == (m // TPAD))


def _pair_e(p):
    mr = lax.broadcasted_iota(_i32, (2 * TPAD, 2 * D), 0)
    dd = lax.broadcasted_iota(_i32, (2 * TPAD, 2 * D), 1)
    n_of = 2 * p + dd // D
    cond = ((mr % TPAD) == n_of) & (((dd % D) // H2) == (mr // TPAD))
    return _bf(cond)


def _attn(nodef, nb, neigh, tidx, msk, te, wq, wk, wv, f1, b1, f2, b2, vs):
    te16 = _bf(te)
    tk = _dot(te16, _bf(wk[D:]))
    tv = _dot(te16, _bf(wv[D:]))
    tq = _dot(te16[0:1], _bf(wq[D:]))
    nodef16 = _bf(nodef)
    q = _dot(nodef16, _bf(wq[:D])) + tq
    qs16 = _bf(q * 0.125)
    wkv16 = _bf(jnp.concatenate([wk[:D], wv[:D]], axis=1))
    tktv16 = jnp.concatenate([_bf(tk), _bf(tv)], axis=1)
    ohi = lax.broadcasted_iota(_i32, (nb, TPAD), 1)
    lane = lax.broadcasted_iota(_i32, (1, 2 * TPAD), 1)
    pen = jnp.where((lane % TPAD) < NN, 0.0, -1e9).astype(_f32)
    z = jnp.zeros((nb, TPAD - NN), _f32)
    mcat = jnp.concatenate([msk, z, msk, z], axis=1)

    s01 = jnp.zeros((nb, 2 * TPAD), _f32)
    for n in range(NN):
        oh16 = _bf(ohi == tidx[:, n:n + 1])
        kv = _dot(_bf(neigh(n)), wkv16) + _dot(oh16, tktv16)
        vs[pl.ds(n * nb, nb)] = kv[:, D:]
        s01 = s01 + _dot(qs16 * _bf(kv[:, :D]), _sel_p(n))

    e = jnp.exp(jnp.where(mcat > 0.5, s01, -40.0) + pen)
    d = _dot(_bf(e), _dsel())
    w16 = _bf(e * (1.0 / d))

    acc = jnp.zeros((nb, D), _f32)
    for p in range(NN // 2):
        wh = _dot(w16, _pair_e(p))
        acc = (acc + wh[:, :D] * vs[pl.ds((2 * p) * nb, nb)]
               + wh[:, D:] * vs[pl.ds((2 * p + 1) * nb, nb)])
    h = jnp.maximum(_dot(_bf(acc), _bf(f1[:D]))
                    + _dot(nodef16, _bf(f1[D:])) + b1, 0.0)
    return _dot(_bf(h), _bf(f2)) + b2


def _bc_f32(x):
    return lax.bitcast_convert_type(x, _f32)


def _tc_body(eu, ei, un, pi, pu, ts, t2p,
             te, wq0, wk0, wv0, f10, b10, f20, b20,
             wq1, wk1, wv1, f11, b11, f21, b21,
             out, o1, vs1, vs2):
    tev = te[...]
    piv = pi[...]
    tidx1 = _tenc(t2p[...] - piv[:, 50:60])
    msk1 = _bc_f32(piv[:, 60:70])
    o1[...] = _attn(ei[...], NB1, lambda n: eu[n], tidx1, msk1, tev,
                    wq0[...], wk0[...], wv0[...], f10[...], b10[...],
                    f20[...], b20[...], vs1)
    puv = pu[...]
    tidx2 = _tenc(ts[...] - puv[:, 20:30])
    msk2 = _bc_f32(puv[:, 30:40])
    out[...] = _attn(un[...], NB2, lambda n: o1[n * NB2:(n + 1) * NB2],
                     tidx2, msk2, tev,
                     wq1[...], wk1[...], wv1[...], f11[...], b11[...],
                     f21[...], b21[...], vs2)


def _tc_specs(nblk):
    full = lambda shape: pl.BlockSpec(shape, lambda i: (0,) * len(shape))
    w = [full((2 * D, D)), full((2 * D, D)), full((2 * D, D)),
         full((2 * D, D)), full((1, D)), full((D, D)), full((1, D))]
    in_specs = [
        pl.BlockSpec((NN, NB1, D), lambda i: (0, i, 0)),
        pl.BlockSpec((NB1, D), lambda i: (i, 0)),
        pl.BlockSpec((NB2, D), lambda i: (i, 0)),
        pl.BlockSpec((NB1, D), lambda i: (i, 0)),
        pl.BlockSpec((NB2, D), lambda i: (i, 0)),
        pl.BlockSpec((NB2, 1), lambda i: (i, 0)),
        pl.BlockSpec((NB1, 1), lambda i: (i, 0)),
        full((TPAD, D)),
    ] + w + w
    out_specs = pl.BlockSpec((NB2, D), lambda i: (i, 0))
    out_shape = jax.ShapeDtypeStruct((nblk * NB2, D), _f32)
    scratch = [pltpu.VMEM((NB1, D), _f32),
               pltpu.VMEM((NN * NB1, D), _f32),
               pltpu.VMEM((NN * NB2, D), _f32)]
    return dict(grid=(nblk,), in_specs=in_specs, out_specs=out_specs,
                out_shape=out_shape, scratch_shapes=scratch)


PBLK = 1000


def _pack_body(un_, ue_, ut_, um_, in_, it_, im_, out):
    t = lambda r: r[...][:, -NN:]
    bi = lambda x: lax.bitcast_convert_type(x, _i32)
    out[...] = jnp.concatenate(
        [t(un_), t(ue_), t(ut_), bi(t(um_)), t(in_), t(it_), bi(t(im_)),
         jnp.zeros((PBLK, D - 7 * NN), _i32)], axis=1)


def _pack(ut, it, ne):
    ncol = ut['neig'].shape[1]
    bs = lambda: pl.BlockSpec((PBLK, ncol), lambda i: (i, 0))
    f = pl.pallas_call(
        _pack_body,
        grid=(ne // PBLK,),
        in_specs=[bs() for _ in range(7)],
        out_specs=pl.BlockSpec((PBLK, D), lambda i: (i, 0)),
        out_shape=jax.ShapeDtypeStruct((ne, D), _i32),
    )
    return f(ut['neig'], ut['edge'], ut['time'], ut['mask'],
             it['neig'], it['time'], it['mask'])


def _prep_params(p):
    return (p['Wq'], p['Wk'], p['Wv'], p['fc1_w'],
            p['fc1_b'].reshape(1, D), p['fc2_w'], p['fc2_b'].reshape(1, D))


def kernel(nodes, edges, timestamps, n_layers, user_emb, item_emb, time_emb,
           tables, layer_params):
    del n_layers
    B = nodes.shape[0]
    nblk = B // NB2
    ut, it = tables['user'], tables['item']
    ne = ut['neig'].shape[0]

    ptab = _pack(ut, it, ne)

    pu_rows, un_nodes = _sc_ga(ptab, user_emb, edges, nodes)

    def perm(x):
        return x.reshape(nblk, NB2, NN).transpose(0, 2, 1).reshape(-1)

    adj2_p = perm(pu_rows[:, 0:NN])
    adge2_p = perm(pu_rows[:, NN:2 * NN])
    t2p = perm(pu_rows[:, 2 * NN:3 * NN]).reshape(-1, 1)
    ei, pi_rows = _sc_gb(item_emb, ptab, adj2_p, adge2_p)

    idx3 = pi_rows[:, 4 * NN:5 * NN].T.reshape(-1)
    eu = _sc_gc(user_emb, idx3).reshape(NN, B * NN, D)

    te = jnp.zeros((TPAD, D), _f32).at[:20].set(time_emb)
    p0 = _prep_params(layer_params[0])
    p1 = _prep_params(layer_params[1])
    tc = pl.pallas_call(_tc_body, **_tc_specs(nblk))
    return tc(eu, ei, un_nodes, pi_rows, pu_rows, timestamps.reshape(-1, 1),
              t2p, te, *p0, *p1)


# speedup vs baseline: 125.6567x; 1.0282x over previous
import jax
import jax.numpy as jnp
from jax import lax
from jax.experimental import pallas as pl
from jax.experimental.pallas import tpu as pltpu
from jax.experimental.pallas import tpu_sc as plsc

D = 128
H2 = D // 2
NN = 10
TPAD = 32
NC, NS = 2, 16
NW = NC * NS
NB2 = 256
NB1 = NB2 * NN

_f32 = jnp.float32
_i32 = jnp.int32


def _mesh():
    return plsc.VectorSubcoreMesh(core_axis_name="c", subcore_axis_name="s")


def _wid():
    return lax.axis_index("s") * NC + lax.axis_index("c")


def _ga_body(ptab, uemb, idxe, idxn, o_pu, o_un, ixe, ixn, bp, bu, sem):
    bpw = 4096 // NW
    base = _wid() * bpw
    pltpu.sync_copy(idxe.at[pl.ds(base, bpw)], ixe)
    pltpu.sync_copy(idxn.at[pl.ds(base, bpw)], ixn)
    pltpu.async_copy(ptab.at[ixe], bp, sem).wait()
    pltpu.sync_copy(bp, o_pu.at[pl.ds(base, bpw)])
    pltpu.async_copy(uemb.at[ixn], bu, sem).wait()
    pltpu.sync_copy(bu, o_un.at[pl.ds(base, bpw)])


def _sc_ga(ptab, uemb, idxe, idxn):
    f = pl.kernel(
        _ga_body,
        out_type=[
            jax.ShapeDtypeStruct((4096, D), _i32),
            jax.ShapeDtypeStruct((4096, D), _f32),
        ],
        mesh=_mesh(),
        scratch_types=[
            pltpu.VMEM((4096 // NW,), _i32),
            pltpu.VMEM((4096 // NW,), _i32),
            pltpu.VMEM((128, D), _i32),
            pltpu.VMEM((128, D), _f32),
            pltpu.SemaphoreType.DMA,
        ],
    )
    return f(ptab, uemb, idxe, idxn)


def _gb_body(iemb, ptab, idxa, idxp, o_ei, o_pi, ixa, ixp, bf, bi,
             sema, semb):
    bpw = 40960 // NW
    ch = 128
    base = _wid() * bpw
    pltpu.sync_copy(idxa.at[pl.ds(base, bpw)], ixa)
    pltpu.sync_copy(idxp.at[pl.ds(base, bpw)], ixp)

    def step(c, carry):
        off = c * ch
        ca = pltpu.async_copy(iemb.at[ixa.at[pl.ds(off, ch)]], bf, sema)
        cb = pltpu.async_copy(ptab.at[ixp.at[pl.ds(off, ch)]], bi, semb)
        ca.wait()
        pltpu.sync_copy(bf, o_ei.at[pl.ds(base + off, ch)])
        cb.wait()
        pltpu.sync_copy(bi, o_pi.at[pl.ds(base + off, ch)])
        return carry

    lax.fori_loop(0, bpw // ch, step, 0)


def _sc_gb(iemb, ptab, idxa, idxp):
    f = pl.kernel(
        _gb_body,
        out_type=[
            jax.ShapeDtypeStruct((40960, D), _f32),
            jax.ShapeDtypeStruct((40960, D), _i32),
        ],
        mesh=_mesh(),
        scratch_types=[
            pltpu.VMEM((40960 // NW,), _i32),
            pltpu.VMEM((40960 // NW,), _i32),
            pltpu.VMEM((128, D), _f32),
            pltpu.VMEM((128, D), _i32),
            pltpu.SemaphoreType.DMA,
            pltpu.SemaphoreType.DMA,
        ],
    )
    return f(iemb, ptab, idxa, idxp)


def _gc_body(uemb, idx, o_eu, ixv, b0, b1, s0, s1):
    bpw = 409600 // NW
    ch = 128
    nch = bpw // ch
    base = _wid() * bpw
    pltpu.sync_copy(idx.at[pl.ds(base, bpw)], ixv)
    pltpu.async_copy(uemb.at[ixv.at[pl.ds(0, ch)]], b0, s0)

    def pair(i, carry):
        off0 = (2 * i) * ch
        off1 = off0 + ch
        pltpu.async_copy(uemb.at[ixv.at[pl.ds(off1, ch)]], b1, s1)
        pltpu.make_async_copy(uemb.at[ixv.at[pl.ds(off0, ch)]], b0, s0).wait()
        pltpu.sync_copy(b0, o_eu.at[pl.ds(base + off0, ch)])

        @pl.when(i + 1 < nch // 2)
        def _():
            pltpu.async_copy(uemb.at[ixv.at[pl.ds(off1 + ch, ch)]], b0, s0)

        pltpu.make_async_copy(uemb.at[ixv.at[pl.ds(off1, ch)]], b1, s1).wait()
        pltpu.sync_copy(b1, o_eu.at[pl.ds(base + off1, ch)])
        return carry

    lax.fori_loop(0, nch // 2, pair, 0)


def _sc_gc(uemb, idx):
    f = pl.kernel(
        _gc_body,
        out_type=jax.ShapeDtypeStruct((409600, D), _f32),
        mesh=_mesh(),
        scratch_types=[
            pltpu.VMEM((409600 // NW,), _i32),
            pltpu.VMEM((128, D), _f32),
            pltpu.VMEM((128, D), _f32),
            pltpu.SemaphoreType.DMA,
            pltpu.SemaphoreType.DMA,
        ],
    )
    return f(uemb, idx)


def _tenc(delta):
    return jnp.clip(jnp.floor(delta.astype(_f32) / 5000.0), 0.0, 19.0).astype(_i32)


def _bf(x):
    return x.astype(jnp.bfloat16)


def _dot(a16, b16):
    return lax.dot_general(a16, b16, (((1,), (0,)), ((), ())),
                           preferred_element_type=_f32)


def _sel_p(n):
    rr = lax.broadcasted_iota(_i32, (D, 2 * TPAD), 0)
    mc = lax.broadcasted_iota(_i32, (D, 2 * TPAD), 1)
    cond = ((mc % TPAD) == n) & ((rr // H2) == (mc // TPAD))
    return _bf(cond)


def _dsel():
    l = lax.broadcasted_iota(_i32, (2 * TPAD, 2 * TPAD), 0)
    m = lax.broadcasted_iota(_i32, (2 * TPAD, 2 * TPAD), 1)
    return _bf((l // TPAD) == (m // TPAD))


def _pair_e(p):
    mr = lax.broadcasted_iota(_i32, (2 * TPAD, 2 * D), 0)
    dd = lax.broadcasted_iota(_i32, (2 * TPAD, 2 * D), 1)
    n_of = 2 * p + dd // D
    cond = ((mr % TPAD) == n_of) & (((dd % D) // H2) == (mr // TPAD))
    return _bf(cond)


def _attn(nodef, nb, neigh, tidx, msk, te, wq, wk, wv, f1, b1, f2, b2, vs):
    te16 = _bf(te)
    tk = _dot(te16, _bf(wk[D:]))
    tv = _dot(te16, _bf(wv[D:]))
    tq = _dot(te16[0:1], _bf(wq[D:]))
    nodef16 = _bf(nodef)
    q = _dot(nodef16, _bf(wq[:D])) + tq
    qs16 = _bf(q * 0.125)
    wkv16 = _bf(jnp.concatenate([wk[:D], wv[:D]], axis=1))
    tktv16 = jnp.concatenate([_bf(tk), _bf(tv)], axis=1)
    ohi = lax.broadcasted_iota(_i32, (nb, TPAD), 1)
    lane = lax.broadcasted_iota(_i32, (1, 2 * TPAD), 1)
    pen = jnp.where((lane % TPAD) < NN, 0.0, -1e9).astype(_f32)
    z = jnp.zeros((nb, TPAD - NN), _f32)
    mcat = jnp.concatenate([msk, z, msk, z], axis=1)

    s01 = jnp.zeros((nb, 2 * TPAD), _f32)
    for n in range(NN):
        oh16 = _bf(ohi == tidx[:, n:n + 1])
        kv = _dot(_bf(neigh(n)), wkv16) + _dot(oh16, tktv16)
        vs[pl.ds(n * nb, nb)] = kv[:, D:]
        s01 = s01 + _dot(qs16 * _bf(kv[:, :D]), _sel_p(n))

    e = jnp.exp(jnp.where(mcat > 0.5, s01, -40.0) + pen)
    d = _dot(_bf(e), _dsel())
    w16 = _bf(e * (1.0 / d))

    acc = jnp.zeros((nb, D), _f32)
    for p in range(NN // 2):
        wh = _dot(w16, _pair_e(p))
        acc = (acc + wh[:, :D] * vs[pl.ds((2 * p) * nb, nb)]
               + wh[:, D:] * vs[pl.ds((2 * p + 1) * nb, nb)])
    h = jnp.maximum(_dot(_bf(acc), _bf(f1[:D]))
                    + _dot(nodef16, _bf(f1[D:])) + b1, 0.0)
    return _dot(_bf(h), _bf(f2)) + b2


def _bc_f32(x):
    return lax.bitcast_convert_type(x, _f32)


def _tc_body(eu, ei, un, pi, pu, ts, t2p,
             te, wq0, wk0, wv0, f10, b10, f20, b20,
             wq1, wk1, wv1, f11, b11, f21, b21,
             out, o1, vs1, vs2):
    tev = te[...]
    piv = pi[...]
    tidx1 = _tenc(t2p[...] - piv[:, 50:60])
    msk1 = _bc_f32(piv[:, 60:70])
    o1[...] = _attn(ei[...], NB1, lambda n: eu[n], tidx1, msk1, tev,
                    wq0[...], wk0[...], wv0[...], f10[...], b10[...],
                    f20[...], b20[...], vs1).reshape(NB2, NN, D)
    puv = pu[...]
    tidx2 = _tenc(ts[...] - puv[:, 20:30])
    msk2 = _bc_f32(puv[:, 30:40])
    out[...] = _attn(un[...], NB2, lambda n: o1[:, n],
                     tidx2, msk2, tev,
                     wq1[...], wk1[...], wv1[...], f11[...], b11[...],
                     f21[...], b21[...], vs2)


def _tc_specs(nblk):
    full = lambda shape: pl.BlockSpec(shape, lambda i: (0,) * len(shape))
    w = [full((2 * D, D)), full((2 * D, D)), full((2 * D, D)),
         full((2 * D, D)), full((1, D)), full((D, D)), full((1, D))]
    in_specs = [
        pl.BlockSpec((NN, NB1, D), lambda i: (0, i, 0)),
        pl.BlockSpec((NB1, D), lambda i: (i, 0)),
        pl.BlockSpec((NB2, D), lambda i: (i, 0)),
        pl.BlockSpec((NB1, D), lambda i: (i, 0)),
        pl.BlockSpec((NB2, D), lambda i: (i, 0)),
        pl.BlockSpec((NB2, 1), lambda i: (i, 0)),
        pl.BlockSpec((NB1, 1), lambda i: (i, 0)),
        full((TPAD, D)),
    ] + w + w
    out_specs = pl.BlockSpec((NB2, D), lambda i: (i, 0))
    out_shape = jax.ShapeDtypeStruct((nblk * NB2, D), _f32)
    scratch = [pltpu.VMEM((NB2, NN, D), _f32),
               pltpu.VMEM((NN * NB1, D), _f32),
               pltpu.VMEM((NN * NB2, D), _f32)]
    return dict(grid=(nblk,), in_specs=in_specs, out_specs=out_specs,
                out_shape=out_shape, scratch_shapes=scratch)


PBLK = 2000


def _pack_body(un_, ue_, ut_, um_, in_, it_, im_, out):
    bi = lambda x: lax.bitcast_convert_type(x[...], _i32)
    out[...] = jnp.concatenate(
        [un_[...], ue_[...], ut_[...], bi(um_), in_[...], it_[...], bi(im_),
         jnp.zeros((PBLK, D - 7 * NN), _i32)], axis=1)


def _pack(ut, it, ne):
    tail = lambda a: a[:, -NN:]
    bs = lambda: pl.BlockSpec((PBLK, NN), lambda i: (i, 0))
    f = pl.pallas_call(
        _pack_body,
        grid=(ne // PBLK,),
        in_specs=[bs() for _ in range(7)],
        out_specs=pl.BlockSpec((PBLK, D), lambda i: (i, 0)),
        out_shape=jax.ShapeDtypeStruct((ne, D), _i32),
    )
    return f(tail(ut['neig']), tail(ut['edge']), tail(ut['time']),
             tail(ut['mask']), tail(it['neig']), tail(it['time']),
             tail(it['mask']))


def _prep_params(p):
    return (p['Wq'], p['Wk'], p['Wv'], p['fc1_w'],
            p['fc1_b'].reshape(1, D), p['fc2_w'], p['fc2_b'].reshape(1, D))


def kernel(nodes, edges, timestamps, n_layers, user_emb, item_emb, time_emb,
           tables, layer_params):
    del n_layers
    B = nodes.shape[0]
    nblk = B // NB2
    ut, it = tables['user'], tables['item']
    ne = ut['neig'].shape[0]

    ptab = _pack(ut, it, ne)

    pu_rows, un_nodes = _sc_ga(ptab, user_emb, edges, nodes)

    adj2_p = pu_rows[:, 0:NN].reshape(-1)
    adge2_p = pu_rows[:, NN:2 * NN].reshape(-1)
    t2p = pu_rows[:, 2 * NN:3 * NN].reshape(-1, 1)
    ei, pi_rows = _sc_gb(item_emb, ptab, adj2_p, adge2_p)

    idx3 = pi_rows[:, 4 * NN:5 * NN].T.reshape(-1)
    eu = _sc_gc(user_emb, idx3).reshape(NN, B * NN, D)

    te = jnp.zeros((TPAD, D), _f32).at[:20].set(time_emb)
    p0 = _prep_params(layer_params[0])
    p1 = _prep_params(layer_params[1])
    tc = pl.pallas_call(_tc_body, **_tc_specs(nblk))
    return tc(eu, ei, un_nodes, pi_rows, pu_rows, timestamps.reshape(-1, 1),
              t2p, te, *p0, *p1)
